# Initial kernel scaffold; baseline (speedup 1.0000x reference)
import jax
import jax.numpy as jnp
from jax.experimental import pallas as pl


def kernel(input_ids, table):
    raise NotImplementedError("write your pallas kernel here")


# speedup vs baseline: 1.6851x; 1.6851x over previous
import functools

import jax
import jax.numpy as jnp
from jax import lax
from jax.experimental import pallas as pl
from jax.experimental.pallas import tpu as pltpu
from jax.experimental.pallas import tpu_sc as plsc


def _make_gather(N, D, CH):
    info = plsc.get_sparse_core_info()
    NC, NS = info.num_cores, info.num_subcores
    NW = NC * NS
    assert N % NW == 0
    b_per_w = N // NW
    assert b_per_w % CH == 0
    nch = b_per_w // CH
    mesh = plsc.VectorSubcoreMesh(core_axis_name="c", subcore_axis_name="s")

    @functools.partial(
        pl.kernel,
        mesh=mesh,
        out_type=jax.ShapeDtypeStruct((N, D), jnp.float32),
        scratch_types=[
            pltpu.VMEM((b_per_w,), jnp.int32),
            pltpu.VMEM((2, CH, D), jnp.float32),
            pltpu.SemaphoreType.DMA,
            pltpu.SemaphoreType.DMA,
            pltpu.SemaphoreType.DMA,
            pltpu.SemaphoreType.DMA,
        ],
    )
    def k(ids_hbm, table_hbm, out_hbm, idx_v, bufs, g0, g1, w0, w1):
        wid = lax.axis_index("s") * NC + lax.axis_index("c")
        base = wid * b_per_w
        pltpu.sync_copy(ids_hbm.at[pl.ds(base, b_per_w)], idx_v)

        gsem = (g0, g1)
        wsem = (w0, w1)
        gather_h = [None, None]
        write_h = [None, None]
        for c in range(nch):
            s = c % 2
            if write_h[s] is not None:
                write_h[s].wait()
            gather_h[s] = pltpu.async_copy(
                table_hbm.at[idx_v.at[pl.ds(c * CH, CH)]], bufs.at[s], gsem[s]
            )
            if c >= 1:
                p = (c - 1) % 2
                gather_h[p].wait()
                write_h[p] = pltpu.async_copy(
                    bufs.at[p], out_hbm.at[pl.ds(base + (c - 1) * CH, CH)], wsem[p]
                )
        p = (nch - 1) % 2
        gather_h[p].wait()
        write_h[p] = pltpu.async_copy(
            bufs.at[p], out_hbm.at[pl.ds(base + (nch - 1) * CH, CH)], wsem[p]
        )
        write_h[0].wait()
        write_h[1].wait()

    return k


def kernel(input_ids, table):
    B, S = input_ids.shape
    V, D = table.shape
    ids = input_ids.reshape(B * S)
    out = _make_gather(B * S, D, 64)(ids, table)
    return out.reshape(B, S, D)
